# baseline (speedup 1.0000x reference)
import jax
import jax.numpy as jnp
from jax import lax
from jax.experimental import pallas as pl
from jax.experimental.pallas import tpu as pltpu
from jax.experimental.pallas import tpu_sc as plsc

B = 16384
C = 1000
NC = 2
NS = 16
NW = NC * NS
RPW = B // NW
GPW = RPW // 16


def _one_hot_body(tgt_hbm, out_hbm, idx_v, stage_v, sem):
    cid = lax.axis_index("c")
    sid = lax.axis_index("s")
    wid = sid * NC + cid
    base = pl.multiple_of(wid * RPW, 8)

    zeros16 = jnp.zeros((16,), jnp.float32)
    ones16 = jnp.ones((16,), jnp.float32)
    lanes = lax.iota(jnp.int32, 16)

    @pl.loop(0, 16)
    def _zr(r):
        @pl.loop(0, 62)
        def _zc(i):
            stage_v[r, pl.ds(pl.multiple_of(i * 16, 16), 16)] = zeros16
        stage_v[r, pl.ds(984, 16)] = zeros16

    pltpu.sync_copy(tgt_hbm.at[pl.ds(base, RPW)], idx_v)

    @pl.loop(0, GPW)
    def _grp(g):
        c16 = idx_v[pl.ds(pl.multiple_of(g * 16, 16), 16)]
        plsc.store_scatter(stage_v, [lanes, c16], ones16)
        pltpu.async_copy(
            stage_v,
            out_hbm.at[pl.ds(pl.multiple_of(base + g * 16, 8), 16)],
            sem).wait()
        plsc.store_scatter(stage_v, [lanes, c16], zeros16)


def kernel(target):
    mesh = plsc.VectorSubcoreMesh(core_axis_name="c", subcore_axis_name="s")
    f = pl.kernel(
        _one_hot_body,
        out_type=jax.ShapeDtypeStruct((B, C), jnp.float32),
        mesh=mesh,
        compiler_params=pltpu.CompilerParams(needs_layout_passes=False),
        scratch_types=[
            pltpu.VMEM((RPW,), jnp.int32),
            pltpu.VMEM((16, C), jnp.float32),
            pltpu.SemaphoreType.DMA,
        ],
    )
    return f(target.astype(jnp.int32))
